# baseline (speedup 1.0000x reference)
import jax
import jax.numpy as jnp
from jax.experimental import pallas as pl

_TOP_K = 2
_SCALE = 2.5


def _gate_body(hs_ref, w_ref, idx_ref, wt_ref):
    hs = hs_ref[...]
    w = w_ref[...]
    logits = jax.lax.dot_general(
        hs, w, (((1,), (1,)), ((), ())), preferred_element_type=jnp.float32
    )
    scores = jax.nn.sigmoid(logits)
    e = scores.shape[1]
    lane = jax.lax.broadcasted_iota(jnp.int32, scores.shape, 1)
    m1 = jnp.max(scores, axis=1, keepdims=True)
    i1 = jnp.min(jnp.where(scores == m1, lane, e), axis=1, keepdims=True)
    masked = jnp.where(lane == i1, -jnp.inf, scores)
    m2 = jnp.max(masked, axis=1, keepdims=True)
    i2 = jnp.min(jnp.where(masked == m2, lane, e), axis=1, keepdims=True)
    denom = m1 + m2 + 1e-20
    idx_ref[...] = jnp.concatenate([i1, i2], axis=1)
    wt_ref[...] = jnp.concatenate([m1 / denom, m2 / denom], axis=1) * _SCALE


def kernel(hidden_states, weight):
    b, s, h = hidden_states.shape
    n = b * s
    e = weight.shape[0]
    hs = hidden_states.reshape(n, h)
    r = 512
    idx, wt = pl.pallas_call(
        _gate_body,
        grid=(n // r,),
        in_specs=[
            pl.BlockSpec((r, h), lambda i: (i, 0)),
            pl.BlockSpec((e, h), lambda i: (0, 0)),
        ],
        out_specs=[
            pl.BlockSpec((r, _TOP_K), lambda i: (i, 0)),
            pl.BlockSpec((r, _TOP_K), lambda i: (i, 0)),
        ],
        out_shape=[
            jax.ShapeDtypeStruct((n, _TOP_K), jnp.int32),
            jax.ShapeDtypeStruct((n, _TOP_K), jnp.float32),
        ],
    )(hs, weight)
    return idx, wt


# speedup vs baseline: 1.2336x; 1.2336x over previous
import jax
import jax.numpy as jnp
from jax.experimental import pallas as pl
from jax.experimental.pallas import tpu as pltpu

_TOP_K = 2
_SCALE = 2.5
_R = 512
_D = 4


def _gate_body(hs_hbm, w_ref, idx_hbm, wt_hbm, buf, idx_acc, wt_acc, sem, osem):
    i = pl.program_id(0)
    nsteps = pl.num_programs(0)

    @pl.when(i == 0)
    def _prime():
        for j in range(_D):
            pltpu.make_async_copy(
                hs_hbm.at[pl.ds(j * _R, _R)], buf.at[j], sem.at[j]
            ).start()

    slot = jax.lax.rem(i, _D)
    pltpu.make_async_copy(
        hs_hbm.at[pl.ds(i * _R, _R)], buf.at[slot], sem.at[slot]
    ).wait()
    hs = buf[slot]

    logits = jax.lax.dot_general(
        hs, w_ref[...], (((1,), (1,)), ((), ())), preferred_element_type=jnp.float32
    )
    scores = jax.nn.sigmoid(logits)
    e = scores.shape[1]
    lane = jax.lax.broadcasted_iota(jnp.int32, scores.shape, 1)
    m1 = jnp.max(scores, axis=1, keepdims=True)
    i1 = jnp.min(jnp.where(scores == m1, lane, e), axis=1, keepdims=True)
    masked = jnp.where(lane == i1, -jnp.inf, scores)
    m2 = jnp.max(masked, axis=1, keepdims=True)
    i2 = jnp.min(jnp.where(masked == m2, lane, e), axis=1, keepdims=True)
    denom = m1 + m2 + 1e-20
    idx_acc[pl.ds(i * _R, _R), :] = jnp.concatenate([i1, i2], axis=1)
    wt_acc[pl.ds(i * _R, _R), :] = (
        jnp.concatenate([m1 / denom, m2 / denom], axis=1) * _SCALE
    )

    nxt = i + _D

    @pl.when(nxt < nsteps)
    def _refill():
        pltpu.make_async_copy(
            hs_hbm.at[pl.ds(nxt * _R, _R)], buf.at[slot], sem.at[slot]
        ).start()

    @pl.when(i == nsteps - 1)
    def _flush():
        pltpu.make_async_copy(idx_acc, idx_hbm, osem.at[0]).start()
        pltpu.make_async_copy(wt_acc, wt_hbm, osem.at[1]).start()
        pltpu.make_async_copy(idx_acc, idx_hbm, osem.at[0]).wait()
        pltpu.make_async_copy(wt_acc, wt_hbm, osem.at[1]).wait()


def kernel(hidden_states, weight):
    b, s, h = hidden_states.shape
    n = b * s
    e = weight.shape[0]
    hs = hidden_states.reshape(n, h)
    idx, wt = pl.pallas_call(
        _gate_body,
        grid=(n // _R,),
        in_specs=[
            pl.BlockSpec(memory_space=pl.ANY),
            pl.BlockSpec((e, h), lambda i: (0, 0)),
        ],
        out_specs=[
            pl.BlockSpec(memory_space=pl.ANY),
            pl.BlockSpec(memory_space=pl.ANY),
        ],
        out_shape=[
            jax.ShapeDtypeStruct((n, _TOP_K), jnp.int32),
            jax.ShapeDtypeStruct((n, _TOP_K), jnp.float32),
        ],
        scratch_shapes=[
            pltpu.VMEM((_D, _R, 2048), jnp.float32),
            pltpu.VMEM((n, _TOP_K), jnp.int32),
            pltpu.VMEM((n, _TOP_K), jnp.float32),
            pltpu.SemaphoreType.DMA((_D,)),
            pltpu.SemaphoreType.DMA((2,)),
        ],
    )(hs, weight)
    return idx, wt


# speedup vs baseline: 1.7549x; 1.4226x over previous
import jax
import jax.numpy as jnp
from jax.experimental import pallas as pl
from jax.experimental.pallas import tpu as pltpu

_TOP_K = 2
_SCALE = 2.5
_R = 512
_D = 4


def _gate_body(hs_hbm, w_ref, idx_hbm, wt_hbm, buf, idx_acc, wt_acc, sem, osem):
    i = pl.program_id(0)
    nsteps = pl.num_programs(0)

    @pl.when(i == 0)
    def _prime():
        for j in range(_D):
            pltpu.make_async_copy(
                hs_hbm.at[pl.ds(j * _R, _R)], buf.at[j], sem.at[j]
            ).start()

    slot = jax.lax.rem(i, _D)
    pltpu.make_async_copy(
        hs_hbm.at[pl.ds(i * _R, _R)], buf.at[slot], sem.at[slot]
    ).wait()
    hs = buf[slot]

    logits_t = jax.lax.dot_general(
        w_ref[...], hs, (((1,), (1,)), ((), ())), preferred_element_type=jnp.float32
    )
    scores = jax.nn.sigmoid(logits_t)
    e = scores.shape[0]
    row = jax.lax.broadcasted_iota(jnp.int32, scores.shape, 0)
    m1 = jnp.max(scores, axis=0)
    i1 = jnp.min(jnp.where(scores == m1[None, :], row, e), axis=0)
    masked = jnp.where(row == i1[None, :], -jnp.inf, scores)
    m2 = jnp.max(masked, axis=0)
    i2 = jnp.min(jnp.where(masked == m2[None, :], row, e), axis=0)
    inv = _SCALE / (m1 + m2 + 1e-20)
    idx_acc[0, pl.ds(i * _R, _R)] = i1
    idx_acc[1, pl.ds(i * _R, _R)] = i2
    wt_acc[0, pl.ds(i * _R, _R)] = m1 * inv
    wt_acc[1, pl.ds(i * _R, _R)] = m2 * inv

    nxt = i + _D

    @pl.when(nxt < nsteps)
    def _refill():
        pltpu.make_async_copy(
            hs_hbm.at[pl.ds(nxt * _R, _R)], buf.at[slot], sem.at[slot]
        ).start()

    @pl.when(i == nsteps - 1)
    def _flush():
        pltpu.make_async_copy(idx_acc, idx_hbm, osem.at[0]).start()
        pltpu.make_async_copy(wt_acc, wt_hbm, osem.at[1]).start()
        pltpu.make_async_copy(idx_acc, idx_hbm, osem.at[0]).wait()
        pltpu.make_async_copy(wt_acc, wt_hbm, osem.at[1]).wait()


def kernel(hidden_states, weight):
    b, s, h = hidden_states.shape
    n = b * s
    e = weight.shape[0]
    hs = hidden_states.reshape(n, h)
    idx_t, wt_t = pl.pallas_call(
        _gate_body,
        grid=(n // _R,),
        in_specs=[
            pl.BlockSpec(memory_space=pl.ANY),
            pl.BlockSpec((e, h), lambda i: (0, 0)),
        ],
        out_specs=[
            pl.BlockSpec(memory_space=pl.ANY),
            pl.BlockSpec(memory_space=pl.ANY),
        ],
        out_shape=[
            jax.ShapeDtypeStruct((_TOP_K, n), jnp.int32),
            jax.ShapeDtypeStruct((_TOP_K, n), jnp.float32),
        ],
        scratch_shapes=[
            pltpu.VMEM((_D, _R, 2048), jnp.float32),
            pltpu.VMEM((_TOP_K, n), jnp.int32),
            pltpu.VMEM((_TOP_K, n), jnp.float32),
            pltpu.SemaphoreType.DMA((_D,)),
            pltpu.SemaphoreType.DMA((2,)),
        ],
    )(hs, weight)
    return idx_t.T, wt_t.T
